# Initial kernel scaffold; baseline (speedup 1.0000x reference)
import jax
import jax.numpy as jnp
from jax.experimental import pallas as pl


def kernel(interior, boundary, batch, edge_index_ii, edge_src_b, edge_dst_b, Wi1, bi1, gi1, bei1, Wi2, bi2, Wb1, bb1, gb1, beb1, Wb2, bb2, Wh1, bh1, Wh2, bh2, Wh3, bh3, Wh4, bh4, Wo1, bo1, go1, beo1, Wo2, bo2, go2, beo2, Wo3, bo3):
    raise NotImplementedError("write your pallas kernel here")


# speedup vs baseline: 4.8004x; 4.8004x over previous
import dataclasses
import functools

import jax
import jax.numpy as jnp
from jax import lax
from jax.experimental import pallas as pl
from jax.experimental.pallas import tpu as pltpu
from jax.experimental.pallas import tpu_sc as plsc

F32 = jnp.float32
N = 10000
BN_ = 2000
NP = 10016
BP = 2016
NW = 32
NSUB = 16
CW = 128
STRIPE = NP // NSUB
EPS = 1e-5

_mesh = plsc.VectorSubcoreMesh(core_axis_name="c", subcore_axis_name="s")
_sc_params = pltpu.CompilerParams()
if "needs_layout_passes" in pltpu.CompilerParams.__dataclass_fields__:
    _sc_params = dataclasses.replace(_sc_params, needs_layout_passes=False)
if "use_tc_tiling_on_sc" in pltpu.CompilerParams.__dataclass_fields__:
    _sc_params = dataclasses.replace(_sc_params, use_tc_tiling_on_sc=False)


def _sds(shape, dtype=F32):
    return jax.ShapeDtypeStruct(shape, dtype)


def _tc_call(fn, out_shape, *args):
    return pl.pallas_call(fn, out_shape=out_shape)(*args)


def _k0_tables(int_ref, bnd_ref, wi1_ref, bi1_ref, wb1_ref, bb1_ref,
               pd_ref, ps_ref, qd_ref, qs_ref):
    x = int_ref[...]
    bd = bnd_ref[...]
    wi = wi1_ref[...]
    wb = wb1_ref[...]
    z = jnp.zeros((NP, 64), F32)
    pd_ref[...] = z
    ps_ref[...] = z
    qd_ref[...] = z
    qs_ref[...] = jnp.zeros((BP, 64), F32)
    dot = functools.partial(jnp.dot, preferred_element_type=F32)
    pd_ref[pl.ds(0, N), :] = dot(x, wi[:3] - wi[3:])
    ps_ref[pl.ds(0, N), :] = dot(x, wi[3:]) + bi1_ref[...]
    qd_ref[pl.ds(0, N), :] = dot(x, wb[:3])
    qs_ref[pl.ds(0, BN_), :] = dot(bd, wb[3:]) + bb1_ref[...]


def _k1b_finalize(stats_ref, cntii_ref, cntib_ref, gi1_ref, bei1_ref,
                  gb1_ref, beb1_ref, aff_ref, cii_ref, cib_ref):
    st = jnp.sum(stats_ref[...], axis=0)
    eii = jnp.float32(160000.0)
    eib = jnp.float32(64000.0)
    m_ii = st[0] / eii
    v_ii = st[1] / eii - m_ii * m_ii
    m_ib = st[2] / eib
    v_ib = st[3] / eib - m_ib * m_ib
    sc_ii = gi1_ref[...] * lax.rsqrt(v_ii + EPS)
    sc_ib = gb1_ref[...] * lax.rsqrt(v_ib + EPS)
    aff_ref[...] = jnp.stack([sc_ii, bei1_ref[...] - m_ii * sc_ii,
                              sc_ib, beb1_ref[...] - m_ib * sc_ib])
    cii_ref[...] = jnp.sum(cntii_ref[...], axis=0)
    cib_ref[...] = jnp.sum(cntib_ref[...], axis=0)


def _k3_x0(uii_ref, uib_ref, cii_ref, cib_ref, wi2_ref, bi2_ref,
           wb2_ref, bb2_ref, xr0_ref):
    dot = functools.partial(jnp.dot, preferred_element_type=F32)
    uii = uii_ref[0] + uii_ref[1]
    uib = uib_ref[0] + uib_ref[1]
    cii = cii_ref[...][:, None]
    cib = cib_ref[...][:, None]
    s = dot(uii, wi2_ref[...]) + cii * bi2_ref[...] \
        + dot(uib, wb2_ref[...]) + cib * bb2_ref[...]
    x0 = s / jnp.maximum(cii + cib, 1.0)
    rowmask = (lax.broadcasted_iota(jnp.int32, (NP, 1), 0) < N).astype(F32)
    xr0_ref[...] = jnp.maximum(x0, 0.0) * rowmask


def _k4b_combine(apart_ref, xr_ref, cii_ref, w_ref, b_ref, xrn_ref, xraw_ref):
    dot = functools.partial(jnp.dot, preferred_element_type=F32)
    s = apart_ref[0] + apart_ref[1]
    cnt = cii_ref[...][:, None]
    deg = jnp.maximum(cnt, 1.0)
    mask = (cnt > 0).astype(F32)
    w = w_ref[...]
    xr = xr_ref[...]
    x = mask * (dot(xr, w[:64] - w[64:]) + b_ref[...]) + dot(s / deg, w[64:])
    rowmask = (lax.broadcasted_iota(jnp.int32, (NP, 1), 0) < N).astype(F32)
    x = x * rowmask
    xraw_ref[...] = x
    xrn_ref[...] = jnp.maximum(x, 0.0)


def _k8_head(f0, f1, f2, f3, f4, wo1_ref, bo1_ref, go1_ref, beo1_ref,
             wo2_ref, bo2_ref, go2_ref, beo2_ref, wo3_ref, bo3_ref, out_ref):
    dot = functools.partial(jnp.dot, preferred_element_type=F32)
    concat = jnp.concatenate(
        [f0[...], f1[...], f2[...], f3[...], f4[...]], axis=1)
    rowmask = (lax.broadcasted_iota(jnp.int32, (NP, 1), 0) < N).astype(F32)
    inv_n = jnp.float32(1.0 / N)

    def bn_relu(h, g, b):
        m = jnp.sum(h * rowmask, axis=0, keepdims=True) * inv_n
        d = (h - m) * rowmask
        v = jnp.sum(d * d, axis=0, keepdims=True) * inv_n
        return jnp.maximum((h - m) * lax.rsqrt(v + EPS) * g + b, 0.0)

    h = bn_relu(dot(concat, wo1_ref[...]) + bo1_ref[...],
                go1_ref[...], beo1_ref[...])
    h = bn_relu(dot(h, wo2_ref[...]) + bo2_ref[...],
                go2_ref[...], beo2_ref[...])
    z = dot(h, wo3_ref[...]) + bo3_ref[...]
    out_ref[...] = (1.0 / (1.0 + jnp.exp(-z)))[:N]


def _sc_stats(nch_ii, nch_ib):
    @functools.partial(
        pl.kernel,
        out_type=(_sds((NW, 4, 64)), _sds((NW, NP)), _sds((NW, NP))),
        mesh=_mesh,
        compiler_params=_sc_params,
        scratch_types=[
            pltpu.VMEM((nch_ii, CW), jnp.int32),
            pltpu.VMEM((nch_ii, CW), jnp.int32),
            pltpu.VMEM((nch_ib, CW), jnp.int32),
            pltpu.VMEM((nch_ib, CW), jnp.int32),
            pltpu.VMEM((CW, 64), F32),
            pltpu.VMEM((CW, 64), F32),
            pltpu.VMEM((NP,), F32),
            pltpu.VMEM((NP,), F32),
            pltpu.VMEM((4, 64), F32),
            pltpu.SemaphoreType.DMA,
            pltpu.SemaphoreType.DMA,
        ],
    )
    def k(pd_hbm, ps_hbm, qd_hbm, qs_hbm, dsti_hbm, srci_hbm, dstb_hbm,
          srcb_hbm, stats_hbm, cntii_hbm, cntib_hbm,
          di_v, si_v, db_v, sb_v, rows_a, rows_b, cii_v, cib_v, stat_v,
          sem1, sem2):
        cid = lax.axis_index("c")
        sid = lax.axis_index("s")
        wid = cid * NSUB + sid
        pltpu.sync_copy(dsti_hbm.at[wid], di_v)
        pltpu.sync_copy(srci_hbm.at[wid], si_v)
        pltpu.sync_copy(dstb_hbm.at[wid], db_v)
        pltpu.sync_copy(srcb_hbm.at[wid], sb_v)
        zero16 = jnp.zeros((16,), F32)

        @pl.loop(0, NP // 16)
        def _(i):
            cii_v[pl.ds(i * 16, 16)] = zero16
            cib_v[pl.ds(i * 16, 16)] = zero16

        ones16 = jnp.ones((16,), F32)

        def process(nch, d_v, s_v, tbl_d, tbl_s, cnt_v):
            def chunk_body(c, accs):
                cp1 = pltpu.async_copy(tbl_d.at[d_v.at[c]], rows_a, sem1)
                cp2 = pltpu.async_copy(tbl_s.at[s_v.at[c]], rows_b, sem2)
                cp1.wait()
                cp2.wait()

                @pl.loop(0, CW // 16)
                def _(g):
                    idx16 = d_v[c, pl.ds(g * 16, 16)]
                    plsc.addupdate_scatter(cnt_v, [idx16], ones16)

                def row_body(r, a8):
                    a8 = list(a8)
                    for q in range(4):
                        sl = pl.ds(q * 16, 16)
                        h = rows_a[r, sl] + rows_b[r, sl]
                        a8[q] = a8[q] + h
                        a8[4 + q] = a8[4 + q] + h * h
                    return tuple(a8)

                return lax.fori_loop(0, CW, row_body, accs)

            init = tuple(jnp.zeros((16,), F32) for _ in range(8))
            return lax.fori_loop(0, nch, chunk_body, init)

        acc_ii = process(nch_ii, di_v, si_v, pd_hbm, ps_hbm, cii_v)
        acc_ib = process(nch_ib, db_v, sb_v, qd_hbm, qs_hbm, cib_v)
        for q in range(4):
            sl = pl.ds(q * 16, 16)
            stat_v[0, sl] = acc_ii[q]
            stat_v[1, sl] = acc_ii[4 + q]
            stat_v[2, sl] = acc_ib[q]
            stat_v[3, sl] = acc_ib[4 + q]
        pltpu.sync_copy(stat_v, stats_hbm.at[wid])
        pltpu.sync_copy(cii_v, cntii_hbm.at[wid])
        pltpu.sync_copy(cib_v, cntib_hbm.at[wid])

    return k


def _sc_apply_scatter(nch_ii, nch_ib):
    @functools.partial(
        pl.kernel,
        out_type=(_sds((2, NP, 64)), _sds((2, NP, 64))),
        mesh=_mesh,
        compiler_params=_sc_params,
        scratch_types=[
            pltpu.VMEM((nch_ii, CW), jnp.int32),
            pltpu.VMEM((nch_ii, CW), jnp.int32),
            pltpu.VMEM((nch_ib, CW), jnp.int32),
            pltpu.VMEM((nch_ib, CW), jnp.int32),
            pltpu.VMEM((CW, 64), F32),
            pltpu.VMEM((CW, 64), F32),
            pltpu.VMEM((4, 64), F32),
            pltpu.VMEM_SHARED((NP, 64), F32),
            pltpu.VMEM_SHARED((NP, 64), F32),
            pltpu.SemaphoreType.DMA,
            pltpu.SemaphoreType.DMA,
        ],
    )
    def k(pd_hbm, ps_hbm, qd_hbm, qs_hbm, dsti_hbm, srci_hbm, dstb_hbm,
          srcb_hbm, aff_hbm, zeros_hbm, uii_hbm, uib_hbm,
          di_v, si_v, db_v, sb_v, rows_a, rows_b, aff_v, sh_u, sh_v,
          sem1, sem2):
        cid = lax.axis_index("c")
        sid = lax.axis_index("s")
        wid = cid * NSUB + sid
        pltpu.sync_copy(dsti_hbm.at[wid], di_v)
        pltpu.sync_copy(srci_hbm.at[wid], si_v)
        pltpu.sync_copy(dstb_hbm.at[wid], db_v)
        pltpu.sync_copy(srcb_hbm.at[wid], sb_v)
        pltpu.sync_copy(aff_hbm, aff_v)
        stripe = pl.ds(sid * STRIPE, STRIPE)
        pltpu.sync_copy(zeros_hbm.at[stripe], sh_u.at[stripe])
        pltpu.sync_copy(zeros_hbm.at[stripe], sh_v.at[stripe])
        plsc.subcore_barrier()

        def process(nch, d_v, s_v, tbl_d, tbl_s, sh, t):
            scale = [aff_v[2 * t, pl.ds(q * 16, 16)] for q in range(4)]
            off = [aff_v[2 * t + 1, pl.ds(q * 16, 16)] for q in range(4)]

            @pl.loop(0, nch)
            def _(c):
                cp1 = pltpu.async_copy(tbl_d.at[d_v.at[c]], rows_a, sem1)
                cp2 = pltpu.async_copy(tbl_s.at[s_v.at[c]], rows_b, sem2)
                cp1.wait()
                cp2.wait()

                @pl.loop(0, CW)
                def _(r):
                    for q in range(4):
                        sl = pl.ds(q * 16, 16)
                        h = (rows_a[r, sl] + rows_b[r, sl]) * scale[q] + off[q]
                        rows_a[r, sl] = jnp.maximum(h, 0.0)

                pltpu.sync_copy(rows_a, sh.at[d_v.at[c]], add=True)

        process(nch_ii, di_v, si_v, pd_hbm, ps_hbm, sh_u, 0)
        process(nch_ib, db_v, sb_v, qd_hbm, qs_hbm, sh_v, 1)
        plsc.subcore_barrier()
        pltpu.sync_copy(sh_u.at[stripe], uii_hbm.at[cid].at[stripe])
        pltpu.sync_copy(sh_v.at[stripe], uib_hbm.at[cid].at[stripe])

    return k


def _sc_segsum(nch_ii):
    @functools.partial(
        pl.kernel,
        out_type=_sds((2, NP, 64)),
        mesh=_mesh,
        compiler_params=_sc_params,
        scratch_types=[
            pltpu.VMEM((nch_ii, CW), jnp.int32),
            pltpu.VMEM((nch_ii, CW), jnp.int32),
            pltpu.VMEM((CW, 64), F32),
            pltpu.VMEM_SHARED((NP, 64), F32),
            pltpu.SemaphoreType.DMA,
        ],
    )
    def k(tbl_hbm, dsti_hbm, srci_hbm, zeros_hbm, out_hbm,
          di_v, si_v, rows, sh_a, sem1):
        cid = lax.axis_index("c")
        sid = lax.axis_index("s")
        wid = cid * NSUB + sid
        pltpu.sync_copy(dsti_hbm.at[wid], di_v)
        pltpu.sync_copy(srci_hbm.at[wid], si_v)
        stripe = pl.ds(sid * STRIPE, STRIPE)
        pltpu.sync_copy(zeros_hbm.at[stripe], sh_a.at[stripe])
        plsc.subcore_barrier()

        @pl.loop(0, nch_ii)
        def _(c):
            pltpu.async_copy(tbl_hbm.at[si_v.at[c]], rows, sem1).wait()
            pltpu.sync_copy(rows, sh_a.at[di_v.at[c]], add=True)

        plsc.subcore_barrier()
        pltpu.sync_copy(sh_a.at[stripe], out_hbm.at[cid].at[stripe])

    return k


def kernel(interior, boundary, batch, edge_index_ii, edge_src_b, edge_dst_b,
           Wi1, bi1, gi1, bei1, Wi2, bi2, Wb1, bb1, gb1, beb1, Wb2, bb2,
           Wh1, bh1, Wh2, bh2, Wh3, bh3, Wh4, bh4,
           Wo1, bo1, go1, beo1, Wo2, bo2, go2, beo2, Wo3, bo3):
    eii = edge_index_ii.shape[1]
    eib = edge_src_b.shape[0]
    nch_ii = -(-eii // (NW * CW))
    nch_ib = -(-eib // (NW * CW))

    def pad_idx(a, nch, fill):
        total = NW * nch * CW
        a = jnp.concatenate(
            [a, jnp.full((total - a.shape[0],), fill, jnp.int32)])
        return a.reshape(NW, nch, CW)

    dsti = pad_idx(edge_index_ii[1], nch_ii, N)
    srci = pad_idx(edge_index_ii[0], nch_ii, N)
    dstb = pad_idx(edge_dst_b, nch_ib, N)
    srcb = pad_idx(edge_src_b, nch_ib, BN_)
    zeros_tbl = jnp.zeros((NP, 64), F32)

    pd, ps, qd, qs = _tc_call(
        _k0_tables,
        (_sds((NP, 64)), _sds((NP, 64)), _sds((NP, 64)), _sds((BP, 64))),
        interior, boundary, Wi1, bi1, Wb1, bb1)

    stats, cntii_w, cntib_w = _sc_stats(nch_ii, nch_ib)(
        pd, ps, qd, qs, dsti, srci, dstb, srcb)

    aff, cii, cib = _tc_call(
        _k1b_finalize,
        (_sds((4, 64)), _sds((NP,)), _sds((NP,))),
        stats, cntii_w, cntib_w, gi1, bei1, gb1, beb1)

    uii, uib = _sc_apply_scatter(nch_ii, nch_ib)(
        pd, ps, qd, qs, dsti, srci, dstb, srcb, aff, zeros_tbl)

    xr = _tc_call(_k3_x0, _sds((NP, 64)),
                  uii, uib, cii, cib, Wi2, bi2, Wb2, bb2)

    segsum = _sc_segsum(nch_ii)
    feats = []
    xraw = xr
    for w, b in ((Wh1, bh1), (Wh2, bh2), (Wh3, bh3), (Wh4, bh4)):
        feats.append(xr)
        apart = segsum(xr, dsti, srci, zeros_tbl)
        xr, xraw = _tc_call(_k4b_combine, (_sds((NP, 64)), _sds((NP, 64))),
                            apart, xr, cii, w, b)
    feats.append(xraw)

    out = _tc_call(_k8_head, _sds((N, 2)),
                   feats[0], feats[1], feats[2], feats[3], feats[4],
                   Wo1, bo1, go1, beo1, Wo2, bo2, go2, beo2, Wo3, bo3)
    return out
